# baseline (speedup 1.0000x reference)
import functools

import jax
import jax.numpy as jnp
from jax.experimental import pallas as pl
from jax.experimental.pallas import tpu as pltpu

_K = 16
_NG = 8
_R = 128
_C = 256
_NEG = -3.0e38
_PEN = 1.0e9
_EXTRACTED = 3.0e9


def _proj_body(x_ref, ws_ref, bs_ref, wh_ref, bh_ref, s_ref, h_ref, sq_ref):
    xb = x_ref[...]
    s = jnp.dot(xb, ws_ref[...], preferred_element_type=jnp.float32,
                precision=jax.lax.Precision.HIGHEST) + bs_ref[...]
    h = jnp.dot(xb, wh_ref[...], preferred_element_type=jnp.float32) + bh_ref[...]
    s_ref[...] = s
    h_ref[...] = h
    sq_ref[...] = jnp.sum(s * s, axis=1, keepdims=True)


def _project(x, Ws, bs, Wh, bh):
    n, _ = x.shape
    sd = Ws.shape[1]
    pd = Wh.shape[1]
    rb = 1000 if n % 1000 == 0 else _R
    grid = pl.cdiv(n, rb)
    return pl.pallas_call(
        _proj_body,
        grid=(grid,),
        in_specs=[
            pl.BlockSpec((rb, x.shape[1]), lambda i: (i, 0)),
            pl.BlockSpec(memory_space=pltpu.VMEM),
            pl.BlockSpec(memory_space=pltpu.VMEM),
            pl.BlockSpec(memory_space=pltpu.VMEM),
            pl.BlockSpec(memory_space=pltpu.VMEM),
        ],
        out_specs=[
            pl.BlockSpec((rb, sd), lambda i: (i, 0)),
            pl.BlockSpec((rb, pd), lambda i: (i, 0)),
            pl.BlockSpec((rb, 1), lambda i: (i, 0)),
        ],
        out_shape=[
            jax.ShapeDtypeStruct((n, sd), jnp.float32),
            jax.ShapeDtypeStruct((n, pd), jnp.float32),
            jax.ShapeDtypeStruct((n, 1), jnp.float32),
        ],
    )(x, Ws, bs.reshape(1, -1), Wh, bh.reshape(1, -1))


def _gnn_body(meta_ref, x_ref, srt_ref, sqr_ref, br_ref, sc_ref, sqc_ref,
              bc_ref, ht_ref, wo1_ref, wo2a_ref, wo2b_ref, bo2_ref,
              out_ref, dwork, *, pd, np_cols):
    i = pl.program_id(0)
    t0 = meta_ref[i, 0]
    nc = meta_ref[i, 1]
    srt = srt_ref[...]
    sqr = sqr_ref[...]
    brow = br_ref[...]

    def fill(t, _):
        j0 = pl.multiple_of((t0 + t) * _C, _C)
        tl = pl.multiple_of(t * _C, _C)
        scc = sc_ref[pl.ds(j0, _C), :]
        m = jnp.dot(scc, srt, preferred_element_type=jnp.float32,
                    precision=jax.lax.Precision.HIGHEST)
        d = (sqr + sqc_ref[pl.ds(j0, _C), :]) - 2.0 * m
        pen = jnp.where(bc_ref[pl.ds(j0, _C), :] != brow, _PEN, 0.0)
        dwork[pl.ds(tl, _C), :] = d + pen
        return 0

    jax.lax.fori_loop(0, nc, fill, 0, unroll=False)

    cachew = 2 * (np_cols // _C)

    def extract(_, carry):
        mean_acc, max_acc, bv1, bp1, bv2, bp2 = carry
        il = jax.lax.broadcasted_iota(jnp.int32, (cachew, _R), 0)

        def sweep(t, acc):
            h1, h2, cv, cp = acc
            tl = pl.multiple_of(t * _C, _C)
            j0 = pl.multiple_of((t0 + t) * _C, _C)
            d = dwork[pl.ds(tl, _C), :]
            colid = jax.lax.broadcasted_iota(jnp.int32, (_C, _R), 0) + tl
            hch = ht_ref[:, pl.ds(j0, _C)]
            h1 = h1 + jnp.dot(hch, (colid == bp1).astype(jnp.float32),
                              preferred_element_type=jnp.float32)
            h2 = h2 + jnp.dot(hch, (colid == bp2).astype(jnp.float32),
                              preferred_element_type=jnp.float32)
            excl = (d < bv2) | ((d == bv2) & (colid <= bp2))
            dm = jnp.where(excl, _EXTRACTED, d)
            cm1 = jnp.min(dm, axis=0, keepdims=True)
            cp1 = jnp.min(jnp.where(dm == cm1, colid, jnp.int32(2**30)),
                          axis=0, keepdims=True)
            dm2 = jnp.where(colid == cp1, _EXTRACTED, dm)
            cm2 = jnp.min(dm2, axis=0, keepdims=True)
            cp2 = jnp.min(jnp.where(dm2 == cm2, colid, jnp.int32(2**30)),
                          axis=0, keepdims=True)
            cv = jnp.where(il == 2 * t, cm1, jnp.where(il == 2 * t + 1, cm2, cv))
            cp = jnp.where(il == 2 * t, cp1, jnp.where(il == 2 * t + 1, cp2, cp))
            return (h1, h2, cv, cp)

        h1, h2, cv, cp = jax.lax.fori_loop(
            0, nc, sweep,
            (jnp.zeros((pd, _R), jnp.float32),
             jnp.zeros((pd, _R), jnp.float32),
             jnp.full((cachew, _R), 4.0e9, jnp.float32),
             jnp.full((cachew, _R), 2**30, jnp.int32)),
            unroll=False)

        m1 = jnp.min(cv, axis=0, keepdims=True)
        p1 = jnp.min(jnp.where(cv == m1, cp, jnp.int32(2**30)),
                     axis=0, keepdims=True)
        cv2 = jnp.where((cv == m1) & (cp == p1), 4.0e9, cv)
        m2 = jnp.min(cv2, axis=0, keepdims=True)
        p2 = jnp.min(jnp.where(cv2 == m2, cp, jnp.int32(2**30)),
                     axis=0, keepdims=True)

        da1 = jnp.where(bv1 >= 0.5 * _PEN, bv1 - _PEN, bv1)
        w1 = jnp.where(bp1 < 0, 0.0, jnp.exp(-10.0 * da1))
        da2 = jnp.where(bv2 >= 0.5 * _PEN, bv2 - _PEN, bv2)
        w2 = jnp.where(bp2 < 0, 0.0, jnp.exp(-10.0 * da2))
        msg1 = h1 * w1
        msg2 = h2 * w2
        new_max = jnp.where(bp1 < 0, max_acc,
                            jnp.maximum(max_acc, jnp.maximum(msg1, msg2)))
        return (mean_acc + msg1 + msg2, new_max, m1, p1, m2, p2)

    neg = jnp.full((1, _R), -1.0e30, jnp.float32)
    negp = jnp.full((1, _R), -1, jnp.int32)
    mean_acc, max_acc, bv1, bp1, bv2, bp2 = jax.lax.fori_loop(
        0, _K // 2 + 1, extract,
        (jnp.zeros((pd, _R), jnp.float32),
         jnp.full((pd, _R), _NEG, jnp.float32),
         neg, negp, neg, negp))

    out = jnp.dot(x_ref[...], wo1_ref[...], preferred_element_type=jnp.float32)
    out = out + jnp.dot((mean_acc * (1.0 / _K)).T, wo2a_ref[...],
                        preferred_element_type=jnp.float32)
    out = out + jnp.dot(max_acc.T, wo2b_ref[...],
                        preferred_element_type=jnp.float32)
    out_ref[...] = out + bo2_ref[...]


def _gravnet_layer(x, batch, starts, Ws, bs, Wh, bh, Wo1, Wo2, bo2):
    n, dim = x.shape
    pd = Wh.shape[1]
    s, h, sq = _project(x, Ws, bs, Wh, bh)

    nblk = pl.cdiv(n, _R)
    npad = nblk * _R
    ncols = pl.cdiv(n, _C) * _C

    row0 = jnp.arange(nblk, dtype=jnp.int32) * _R
    rowl = jnp.minimum(row0 + _R - 1, n - 1)
    g0 = batch[row0]
    g1 = batch[rowl]
    c0 = starts[g0]
    c1 = starts[g1 + 1]
    t0 = c0 // _C
    t1 = (c1 + _C - 1) // _C
    meta = jnp.stack([t0, jnp.maximum(t1 - t0, 1)], axis=1).astype(jnp.int32)

    xp = jnp.pad(x, ((0, npad - n), (0, 0)))
    scp = jnp.pad(s, ((0, ncols - n), (0, 0)))
    stp = jnp.pad(s.T, ((0, 0), (0, ncols - n)))
    sqvp = jnp.pad(sq, ((0, ncols - n), (0, 0)))
    sqlp = jnp.pad(sq.T, ((0, 0), (0, ncols - n)))
    bvp = jnp.pad(batch.reshape(n, 1), ((0, ncols - n), (0, 0)),
                  constant_values=-1)
    blp = jnp.pad(batch.reshape(1, n), ((0, 0), (0, ncols - n)),
                  constant_values=-2)
    htp = jnp.pad(h.T, ((0, 0), (0, ncols - n)))

    body = functools.partial(_gnn_body, pd=pd, np_cols=ncols)
    out = pl.pallas_call(
        body,
        grid=(nblk,),
        in_specs=[
            pl.BlockSpec(memory_space=pltpu.SMEM),
            pl.BlockSpec((_R, dim), lambda i: (i, 0)),
            pl.BlockSpec((Ws.shape[1], _R), lambda i: (0, i)),
            pl.BlockSpec((1, _R), lambda i: (0, i)),
            pl.BlockSpec((1, _R), lambda i: (0, i)),
            pl.BlockSpec(memory_space=pltpu.VMEM),
            pl.BlockSpec(memory_space=pltpu.VMEM),
            pl.BlockSpec(memory_space=pltpu.VMEM),
            pl.BlockSpec(memory_space=pltpu.VMEM),
            pl.BlockSpec(memory_space=pltpu.VMEM),
            pl.BlockSpec(memory_space=pltpu.VMEM),
            pl.BlockSpec(memory_space=pltpu.VMEM),
            pl.BlockSpec(memory_space=pltpu.VMEM),
        ],
        out_specs=pl.BlockSpec((_R, dim), lambda i: (i, 0)),
        out_shape=jax.ShapeDtypeStruct((npad, dim), jnp.float32),
        scratch_shapes=[pltpu.VMEM((ncols, _R), jnp.float32)],
    )(meta, xp, stp, sqlp, blp, scp, sqvp, bvp, htp,
      Wo1, Wo2[:pd, :], Wo2[pd:, :], bo2.reshape(1, -1))
    return out[:n]


def _beta_body(l_ref, w1_ref, b1_ref, w2_ref, b2_ref, w3_ref, b3_ref, o_ref):
    hb = jnp.maximum(
        jnp.dot(l_ref[...], w1_ref[...], preferred_element_type=jnp.float32)
        + b1_ref[...], 0.0)
    hb = jnp.maximum(
        jnp.dot(hb, w2_ref[...], preferred_element_type=jnp.float32)
        + b2_ref[...], 0.0)
    z = jnp.dot(hb, w3_ref[...], preferred_element_type=jnp.float32) + b3_ref[...]
    beta = 1.0 / (1.0 + jnp.exp(-z))
    o_ref[...] = jnp.clip(beta, 1e-6, 1.0 - 1e-6)


def _beta_mlp(latent, W1, b1, W2, b2, W3, b3):
    n, dim = latent.shape
    rb = 1000 if n % 1000 == 0 else _R
    return pl.pallas_call(
        _beta_body,
        grid=(pl.cdiv(n, rb),),
        in_specs=[pl.BlockSpec((rb, dim), lambda i: (i, 0))]
        + [pl.BlockSpec(memory_space=pltpu.VMEM)] * 6,
        out_specs=pl.BlockSpec((rb, 1), lambda i: (i, 0)),
        out_shape=jax.ShapeDtypeStruct((n, 1), jnp.float32),
    )(latent, W1, b1.reshape(1, -1), W2, b2.reshape(1, -1),
      W3, b3.reshape(1, -1))


def kernel(x, batch, l1_Ws, l1_bs, l1_Wh, l1_bh, l1_Wo1, l1_Wo2, l1_bo2,
           l2_Ws, l2_bs, l2_Wh, l2_bh, l2_Wo1, l2_Wo2, l2_bo2,
           b_W1, b_b1, b_W2, b_b2, b_W3, b_b3):
    batch = batch.astype(jnp.int32)
    starts = jnp.searchsorted(
        batch, jnp.arange(_NG + 1, dtype=jnp.int32), side='left'
    ).astype(jnp.int32)
    latent = _gravnet_layer(x, batch, starts, l1_Ws, l1_bs, l1_Wh, l1_bh,
                            l1_Wo1, l1_Wo2, l1_bo2)
    latent = _gravnet_layer(latent, batch, starts, l2_Ws, l2_bs, l2_Wh, l2_bh,
                            l2_Wo1, l2_Wo2, l2_bo2)
    beta = _beta_mlp(latent, b_W1, b_b1, b_W2, b_b2, b_W3, b_b3)
    return (beta, latent)


# speedup vs baseline: 1.3018x; 1.3018x over previous
import functools

import jax
import jax.numpy as jnp
from jax.experimental import pallas as pl
from jax.experimental.pallas import tpu as pltpu

_K = 16
_NG = 8
_R = 128
_C = 512
_NEG = -3.0e38
_PEN = 1.0e9
_EXTRACTED = 3.0e9


def _proj_body(x_ref, ws_ref, bs_ref, wh_ref, bh_ref, s_ref, h_ref, sq_ref):
    xb = x_ref[...]
    s = jnp.dot(xb, ws_ref[...], preferred_element_type=jnp.float32,
                precision=jax.lax.Precision.HIGHEST) + bs_ref[...]
    h = jnp.dot(xb, wh_ref[...], preferred_element_type=jnp.float32) + bh_ref[...]
    s_ref[...] = s
    h_ref[...] = h
    sq_ref[...] = jnp.sum(s * s, axis=1, keepdims=True)


def _project(x, Ws, bs, Wh, bh):
    n, _ = x.shape
    sd = Ws.shape[1]
    pd = Wh.shape[1]
    rb = 1000 if n % 1000 == 0 else _R
    grid = pl.cdiv(n, rb)
    return pl.pallas_call(
        _proj_body,
        grid=(grid,),
        in_specs=[
            pl.BlockSpec((rb, x.shape[1]), lambda i: (i, 0)),
            pl.BlockSpec(memory_space=pltpu.VMEM),
            pl.BlockSpec(memory_space=pltpu.VMEM),
            pl.BlockSpec(memory_space=pltpu.VMEM),
            pl.BlockSpec(memory_space=pltpu.VMEM),
        ],
        out_specs=[
            pl.BlockSpec((rb, sd), lambda i: (i, 0)),
            pl.BlockSpec((rb, pd), lambda i: (i, 0)),
            pl.BlockSpec((rb, 1), lambda i: (i, 0)),
        ],
        out_shape=[
            jax.ShapeDtypeStruct((n, sd), jnp.float32),
            jax.ShapeDtypeStruct((n, pd), jnp.float32),
            jax.ShapeDtypeStruct((n, 1), jnp.float32),
        ],
    )(x, Ws, bs.reshape(1, -1), Wh, bh.reshape(1, -1))


def _gnn_body(meta_ref, x_ref, srt_ref, sqr_ref, br_ref, sc_ref, sqc_ref,
              bc_ref, ht_ref, wo1_ref, wo2a_ref, wo2b_ref, bo2_ref,
              out_ref, dwork, *, pd, np_cols):
    i = pl.program_id(0)
    t0 = meta_ref[i, 0]
    nc = meta_ref[i, 1]
    srt = srt_ref[...]
    sqr = sqr_ref[...]
    brow = br_ref[...]

    def fill(t, _):
        j0 = pl.multiple_of((t0 + t) * _C, _C)
        tl = pl.multiple_of(t * _C, _C)
        scc = sc_ref[pl.ds(j0, _C), :]
        m = jnp.dot(scc, srt, preferred_element_type=jnp.float32,
                    precision=jax.lax.Precision.HIGHEST)
        d = (sqr + sqc_ref[pl.ds(j0, _C), :]) - 2.0 * m
        pen = jnp.where(bc_ref[pl.ds(j0, _C), :] != brow, _PEN, 0.0)
        dwork[pl.ds(tl, _C), :] = d + pen
        return 0

    jax.lax.fori_loop(0, nc, fill, 0, unroll=False)

    cachew = 2 * (np_cols // _C)

    def extract(_, carry):
        mean_acc, max_acc, bv1, bp1, bv2, bp2 = carry
        il = jax.lax.broadcasted_iota(jnp.int32, (cachew, _R), 0)

        def sweep(t, acc):
            h1, h2, cv, cp = acc
            tl = pl.multiple_of(t * _C, _C)
            j0 = pl.multiple_of((t0 + t) * _C, _C)
            d = dwork[pl.ds(tl, _C), :]
            colid = jax.lax.broadcasted_iota(jnp.int32, (_C, _R), 0) + tl
            hch = ht_ref[:, pl.ds(j0, _C)]
            h1 = h1 + jnp.dot(hch, (colid == bp1).astype(jnp.float32),
                              preferred_element_type=jnp.float32)
            h2 = h2 + jnp.dot(hch, (colid == bp2).astype(jnp.float32),
                              preferred_element_type=jnp.float32)
            excl = (d < bv2) | ((d == bv2) & (colid <= bp2))
            dm = jnp.where(excl, _EXTRACTED, d)
            cm1 = jnp.min(dm, axis=0, keepdims=True)
            cp1 = jnp.min(jnp.where(dm == cm1, colid, jnp.int32(2**30)),
                          axis=0, keepdims=True)
            dm2 = jnp.where(colid == cp1, _EXTRACTED, dm)
            cm2 = jnp.min(dm2, axis=0, keepdims=True)
            cp2 = jnp.min(jnp.where(dm2 == cm2, colid, jnp.int32(2**30)),
                          axis=0, keepdims=True)
            cv = jnp.where(il == 2 * t, cm1, jnp.where(il == 2 * t + 1, cm2, cv))
            cp = jnp.where(il == 2 * t, cp1, jnp.where(il == 2 * t + 1, cp2, cp))
            return (h1, h2, cv, cp)

        h1, h2, cv, cp = jax.lax.fori_loop(
            0, nc, sweep,
            (jnp.zeros((pd, _R), jnp.float32),
             jnp.zeros((pd, _R), jnp.float32),
             jnp.full((cachew, _R), 4.0e9, jnp.float32),
             jnp.full((cachew, _R), 2**30, jnp.int32)),
            unroll=False)

        m1 = jnp.min(cv, axis=0, keepdims=True)
        p1 = jnp.min(jnp.where(cv == m1, cp, jnp.int32(2**30)),
                     axis=0, keepdims=True)
        cv2 = jnp.where((cv == m1) & (cp == p1), 4.0e9, cv)
        m2 = jnp.min(cv2, axis=0, keepdims=True)
        p2 = jnp.min(jnp.where(cv2 == m2, cp, jnp.int32(2**30)),
                     axis=0, keepdims=True)

        da1 = jnp.where(bv1 >= 0.5 * _PEN, bv1 - _PEN, bv1)
        w1 = jnp.where(bp1 < 0, 0.0, jnp.exp(-10.0 * da1))
        da2 = jnp.where(bv2 >= 0.5 * _PEN, bv2 - _PEN, bv2)
        w2 = jnp.where(bp2 < 0, 0.0, jnp.exp(-10.0 * da2))
        msg1 = h1 * w1
        msg2 = h2 * w2
        new_max = jnp.where(bp1 < 0, max_acc,
                            jnp.maximum(max_acc, jnp.maximum(msg1, msg2)))
        return (mean_acc + msg1 + msg2, new_max, m1, p1, m2, p2)

    neg = jnp.full((1, _R), -1.0e30, jnp.float32)
    negp = jnp.full((1, _R), -1, jnp.int32)
    mean_acc, max_acc, bv1, bp1, bv2, bp2 = jax.lax.fori_loop(
        0, _K // 2 + 1, extract,
        (jnp.zeros((pd, _R), jnp.float32),
         jnp.full((pd, _R), _NEG, jnp.float32),
         neg, negp, neg, negp))

    out = jnp.dot(x_ref[...], wo1_ref[...], preferred_element_type=jnp.float32)
    out = out + jnp.dot((mean_acc * (1.0 / _K)).T, wo2a_ref[...],
                        preferred_element_type=jnp.float32)
    out = out + jnp.dot(max_acc.T, wo2b_ref[...],
                        preferred_element_type=jnp.float32)
    out_ref[...] = out + bo2_ref[...]


def _gravnet_layer(x, batch, starts, Ws, bs, Wh, bh, Wo1, Wo2, bo2):
    n, dim = x.shape
    pd = Wh.shape[1]
    s, h, sq = _project(x, Ws, bs, Wh, bh)

    nblk = pl.cdiv(n, _R)
    npad = nblk * _R
    ncols = pl.cdiv(n, _C) * _C

    row0 = jnp.arange(nblk, dtype=jnp.int32) * _R
    rowl = jnp.minimum(row0 + _R - 1, n - 1)
    g0 = batch[row0]
    g1 = batch[rowl]
    c0 = starts[g0]
    c1 = starts[g1 + 1]
    t0 = c0 // _C
    t1 = (c1 + _C - 1) // _C
    meta = jnp.stack([t0, jnp.maximum(t1 - t0, 1)], axis=1).astype(jnp.int32)

    xp = jnp.pad(x, ((0, npad - n), (0, 0)))
    scp = jnp.pad(s, ((0, ncols - n), (0, 0)))
    stp = jnp.pad(s.T, ((0, 0), (0, ncols - n)))
    sqvp = jnp.pad(sq, ((0, ncols - n), (0, 0)))
    sqlp = jnp.pad(sq.T, ((0, 0), (0, ncols - n)))
    bvp = jnp.pad(batch.reshape(n, 1), ((0, ncols - n), (0, 0)),
                  constant_values=-1)
    blp = jnp.pad(batch.reshape(1, n), ((0, 0), (0, ncols - n)),
                  constant_values=-2)
    htp = jnp.pad(h.T, ((0, 0), (0, ncols - n)))

    body = functools.partial(_gnn_body, pd=pd, np_cols=ncols)
    out = pl.pallas_call(
        body,
        grid=(nblk,),
        in_specs=[
            pl.BlockSpec(memory_space=pltpu.SMEM),
            pl.BlockSpec((_R, dim), lambda i: (i, 0)),
            pl.BlockSpec((Ws.shape[1], _R), lambda i: (0, i)),
            pl.BlockSpec((1, _R), lambda i: (0, i)),
            pl.BlockSpec((1, _R), lambda i: (0, i)),
            pl.BlockSpec(memory_space=pltpu.VMEM),
            pl.BlockSpec(memory_space=pltpu.VMEM),
            pl.BlockSpec(memory_space=pltpu.VMEM),
            pl.BlockSpec(memory_space=pltpu.VMEM),
            pl.BlockSpec(memory_space=pltpu.VMEM),
            pl.BlockSpec(memory_space=pltpu.VMEM),
            pl.BlockSpec(memory_space=pltpu.VMEM),
            pl.BlockSpec(memory_space=pltpu.VMEM),
        ],
        out_specs=pl.BlockSpec((_R, dim), lambda i: (i, 0)),
        out_shape=jax.ShapeDtypeStruct((npad, dim), jnp.float32),
        scratch_shapes=[pltpu.VMEM((ncols, _R), jnp.float32)],
    )(meta, xp, stp, sqlp, blp, scp, sqvp, bvp, htp,
      Wo1, Wo2[:pd, :], Wo2[pd:, :], bo2.reshape(1, -1))
    return out[:n]


def _beta_body(l_ref, w1_ref, b1_ref, w2_ref, b2_ref, w3_ref, b3_ref, o_ref):
    hb = jnp.maximum(
        jnp.dot(l_ref[...], w1_ref[...], preferred_element_type=jnp.float32)
        + b1_ref[...], 0.0)
    hb = jnp.maximum(
        jnp.dot(hb, w2_ref[...], preferred_element_type=jnp.float32)
        + b2_ref[...], 0.0)
    z = jnp.dot(hb, w3_ref[...], preferred_element_type=jnp.float32) + b3_ref[...]
    beta = 1.0 / (1.0 + jnp.exp(-z))
    o_ref[...] = jnp.clip(beta, 1e-6, 1.0 - 1e-6)


def _beta_mlp(latent, W1, b1, W2, b2, W3, b3):
    n, dim = latent.shape
    rb = 1000 if n % 1000 == 0 else _R
    return pl.pallas_call(
        _beta_body,
        grid=(pl.cdiv(n, rb),),
        in_specs=[pl.BlockSpec((rb, dim), lambda i: (i, 0))]
        + [pl.BlockSpec(memory_space=pltpu.VMEM)] * 6,
        out_specs=pl.BlockSpec((rb, 1), lambda i: (i, 0)),
        out_shape=jax.ShapeDtypeStruct((n, 1), jnp.float32),
    )(latent, W1, b1.reshape(1, -1), W2, b2.reshape(1, -1),
      W3, b3.reshape(1, -1))


def kernel(x, batch, l1_Ws, l1_bs, l1_Wh, l1_bh, l1_Wo1, l1_Wo2, l1_bo2,
           l2_Ws, l2_bs, l2_Wh, l2_bh, l2_Wo1, l2_Wo2, l2_bo2,
           b_W1, b_b1, b_W2, b_b2, b_W3, b_b3):
    batch = batch.astype(jnp.int32)
    starts = jnp.searchsorted(
        batch, jnp.arange(_NG + 1, dtype=jnp.int32), side='left'
    ).astype(jnp.int32)
    latent = _gravnet_layer(x, batch, starts, l1_Ws, l1_bs, l1_Wh, l1_bh,
                            l1_Wo1, l1_Wo2, l1_bo2)
    latent = _gravnet_layer(latent, batch, starts, l2_Ws, l2_bs, l2_Wh, l2_bh,
                            l2_Wo1, l2_Wo2, l2_bo2)
    beta = _beta_mlp(latent, b_W1, b_b1, b_W2, b_b2, b_W3, b_b3)
    return (beta, latent)
